# Initial kernel scaffold; baseline (speedup 1.0000x reference)
import jax
import jax.numpy as jnp
from jax.experimental import pallas as pl


def kernel(input, graph_indices, node_counts, W1, W2):
    raise NotImplementedError("write your pallas kernel here")


# speedup vs baseline: 2.1007x; 2.1007x over previous
import jax
import jax.numpy as jnp
from jax import lax
from jax.experimental import pallas as pl
from jax.experimental.pallas import tpu as pltpu
from jax.experimental.pallas import tpu_sc as plsc

N = 320000
D = 128
G = 512
NC, NS = 2, 16
NW = NC * NS
ROWS_W = N // NW
CHUNK = 40
NCH = ROWS_W // CHUNK
BM = 512


def _gate_body(x_ref, w_ref, z_ref):
    y = jnp.dot(x_ref[...], w_ref[...], preferred_element_type=jnp.float32)
    a = y[:, :D]
    b = y[:, D:]
    z_ref[...] = jnp.where(a > 0.0, a, jnp.exp(a) - 1.0) * b


def _gated_matmul(x, wc):
    return pl.pallas_call(
        _gate_body,
        grid=(N // BM,),
        in_specs=[
            pl.BlockSpec((BM, D), lambda i: (i, 0)),
            pl.BlockSpec((D, 2 * D), lambda i: (0, 0)),
        ],
        out_specs=pl.BlockSpec((BM, D), lambda i: (i, 0)),
        out_shape=jax.ShapeDtypeStruct((N, D), jnp.float32),
    )(x, wc)


def _sc_body(z_hbm, idx_hbm, zero_hbm, out_hbm,
             idx_v, zb0, zb1, stage, shared, sem0, sem1):
    c = lax.axis_index("c")
    s = lax.axis_index("s")
    wid = c * NS + s
    gs = G // NS
    pltpu.sync_copy(zero_hbm.at[pl.ds(s * gs, gs)], shared.at[pl.ds(s * gs, gs)])
    pltpu.sync_copy(idx_hbm.at[wid], idx_v)
    plsc.subcore_barrier()

    row0 = wid * ROWS_W
    pltpu.make_async_copy(z_hbm.at[pl.ds(row0, CHUNK)], zb0, sem0).start()
    pltpu.make_async_copy(z_hbm.at[pl.ds(row0 + CHUNK, CHUNK)], zb1, sem1).start()

    def step(k, carry):
        j0 = 2 * k
        pltpu.make_async_copy(z_hbm.at[pl.ds(row0 + j0 * CHUNK, CHUNK)],
                              zb0, sem0).wait()
        pltpu.sync_copy(zb0, shared.at[idx_v.at[j0]], add=True)

        @pl.when(j0 + 2 < NCH)
        def _():
            pltpu.make_async_copy(
                z_hbm.at[pl.ds(row0 + (j0 + 2) * CHUNK, CHUNK)], zb0, sem0
            ).start()

        pltpu.make_async_copy(z_hbm.at[pl.ds(row0 + (j0 + 1) * CHUNK, CHUNK)],
                              zb1, sem1).wait()
        pltpu.sync_copy(zb1, shared.at[idx_v.at[j0 + 1]], add=True)

        @pl.when(j0 + 3 < NCH)
        def _():
            pltpu.make_async_copy(
                z_hbm.at[pl.ds(row0 + (j0 + 3) * CHUNK, CHUNK)], zb1, sem1
            ).start()

        return carry

    lax.fori_loop(0, NCH // 2, step, 0)
    plsc.subcore_barrier()
    pltpu.sync_copy(shared.at[pl.ds(s * gs, gs)], stage)
    pltpu.sync_copy(stage, out_hbm.at[c, pl.ds(s * gs, gs)])


def _segment_sum_sc(z, idx3, zeros):
    mesh = plsc.VectorSubcoreMesh(
        core_axis_name="c", subcore_axis_name="s",
        num_cores=NC, num_subcores=NS,
    )
    return pl.kernel(
        _sc_body,
        out_type=jax.ShapeDtypeStruct((NC, G, D), jnp.float32),
        mesh=mesh,
        scratch_types=[
            pltpu.VMEM((NCH, CHUNK), jnp.int32),
            pltpu.VMEM((CHUNK, D), jnp.float32),
            pltpu.VMEM((CHUNK, D), jnp.float32),
            pltpu.VMEM((G // NS, D), jnp.float32),
            pltpu.VMEM_SHARED((G, D), jnp.float32),
            pltpu.SemaphoreType.DMA,
            pltpu.SemaphoreType.DMA,
        ],
    )(z, idx3, zeros)


def _merge_body(p_ref, o_ref):
    o_ref[...] = p_ref[0] + p_ref[1]


def _merge(partials):
    return pl.pallas_call(
        _merge_body,
        out_shape=jax.ShapeDtypeStruct((G, D), jnp.float32),
    )(partials)


def kernel(input, graph_indices, node_counts, W1, W2):
    del node_counts
    wc = jnp.concatenate([W1, W2], axis=0).T
    idx3 = graph_indices.astype(jnp.int32).reshape(NW, NCH, CHUNK)
    zeros = jnp.zeros((G, D), jnp.float32)
    z = _gated_matmul(input, wc)
    partials = _segment_sum_sc(z, idx3, zeros)
    return _merge(partials)


# speedup vs baseline: 2.5166x; 1.1980x over previous
import jax
import jax.numpy as jnp
from jax import lax
from jax.experimental import pallas as pl
from jax.experimental.pallas import tpu as pltpu
from jax.experimental.pallas import tpu_sc as plsc

N = 320000
D = 128
G = 512
S = 5
NSLAB = N // S
NC, NS = 2, 16
NW = NC * NS
ROWS_W = NSLAB // NW
CHUNK = 40
NCH = ROWS_W // CHUNK
BM = 512


def _gate_body(x_ref, w_ref, z_ref):
    y = jnp.dot(x_ref[...], w_ref[...], preferred_element_type=jnp.float32)
    a = y[:, :D]
    b = y[:, D:]
    z_ref[...] = jnp.where(a > 0.0, a, jnp.exp(a) - 1.0) * b


def _gated_matmul(x, wc, slab):
    nblk = NSLAB // BM
    return pl.pallas_call(
        _gate_body,
        grid=(nblk,),
        in_specs=[
            pl.BlockSpec((BM, D), lambda i, s=slab, n=nblk: (s * n + i, 0)),
            pl.BlockSpec((D, 2 * D), lambda i: (0, 0)),
        ],
        out_specs=pl.BlockSpec((BM, D), lambda i: (i, 0)),
        out_shape=jax.ShapeDtypeStruct((NSLAB, D), jnp.float32),
    )(x, wc)


def _sc_body(z_hbm, idx_hbm, zero_hbm, out_hbm,
             idx_v, zb0, zb1, stage, shared, sem0, sem1):
    c = lax.axis_index("c")
    s = lax.axis_index("s")
    wid = c * NS + s
    gs = G // NS
    pltpu.sync_copy(zero_hbm.at[pl.ds(s * gs, gs)], shared.at[pl.ds(s * gs, gs)])
    pltpu.sync_copy(idx_hbm.at[wid], idx_v)
    plsc.subcore_barrier()

    row0 = wid * ROWS_W
    pltpu.make_async_copy(z_hbm.at[pl.ds(row0, CHUNK)], zb0, sem0).start()
    pltpu.make_async_copy(z_hbm.at[pl.ds(row0 + CHUNK, CHUNK)], zb1, sem1).start()

    def step(k, carry):
        j0 = 2 * k
        pltpu.make_async_copy(z_hbm.at[pl.ds(row0 + j0 * CHUNK, CHUNK)],
                              zb0, sem0).wait()
        pltpu.sync_copy(zb0, shared.at[idx_v.at[j0]], add=True)

        @pl.when(j0 + 2 < NCH)
        def _():
            pltpu.make_async_copy(
                z_hbm.at[pl.ds(row0 + (j0 + 2) * CHUNK, CHUNK)], zb0, sem0
            ).start()

        pltpu.make_async_copy(z_hbm.at[pl.ds(row0 + (j0 + 1) * CHUNK, CHUNK)],
                              zb1, sem1).wait()
        pltpu.sync_copy(zb1, shared.at[idx_v.at[j0 + 1]], add=True)

        @pl.when(j0 + 3 < NCH)
        def _():
            pltpu.make_async_copy(
                z_hbm.at[pl.ds(row0 + (j0 + 3) * CHUNK, CHUNK)], zb1, sem1
            ).start()

        return carry

    lax.fori_loop(0, NCH // 2, step, 0)
    plsc.subcore_barrier()
    pltpu.sync_copy(shared.at[pl.ds(s * gs, gs)], stage)
    pltpu.sync_copy(stage, out_hbm.at[c, pl.ds(s * gs, gs)])


def _segment_sum_sc(z, idx3, zeros):
    mesh = plsc.VectorSubcoreMesh(
        core_axis_name="c", subcore_axis_name="s",
        num_cores=NC, num_subcores=NS,
    )
    return pl.kernel(
        _sc_body,
        out_type=jax.ShapeDtypeStruct((NC, G, D), jnp.float32),
        mesh=mesh,
        scratch_types=[
            pltpu.VMEM((NCH, CHUNK), jnp.int32),
            pltpu.VMEM((CHUNK, D), jnp.float32),
            pltpu.VMEM((CHUNK, D), jnp.float32),
            pltpu.VMEM((G // NS, D), jnp.float32),
            pltpu.VMEM_SHARED((G, D), jnp.float32),
            pltpu.SemaphoreType.DMA,
            pltpu.SemaphoreType.DMA,
        ],
    )(z, idx3, zeros)


def _merge_body(p_ref, o_ref):
    o_ref[...] = jnp.sum(p_ref[...], axis=0)


def _merge(partials):
    return pl.pallas_call(
        _merge_body,
        out_shape=jax.ShapeDtypeStruct((G, D), jnp.float32),
    )(partials)


def kernel(input, graph_indices, node_counts, W1, W2):
    del node_counts
    wc = jnp.concatenate([W1, W2], axis=0).T
    idx4 = graph_indices.astype(jnp.int32).reshape(S, NW, NCH, CHUNK)
    zeros = jnp.zeros((G, D), jnp.float32)
    parts = []
    for slab in range(S):
        z = _gated_matmul(input, wc, slab)
        parts.append(_segment_sum_sc(z, idx4[slab], zeros))
    partials = jnp.stack(parts).reshape(S * NC, G, D)
    return _merge(partials)


# speedup vs baseline: 4.5657x; 1.8142x over previous
import jax
import jax.numpy as jnp
from jax import lax
from jax.experimental import pallas as pl
from jax.experimental.pallas import tpu as pltpu
from jax.experimental.pallas import tpu_sc as plsc

N = 320000
D = 128
G = 512
S = 5
NSLAB = N // S
NC, NS = 2, 16
NW = NC * NS
ROWS_W = NSLAB // NW
CHUNK = 40
NCH = ROWS_W // CHUNK
BM = 1600


def _gate_body(x_ref, w_ref, z_ref):
    y = jnp.dot(x_ref[...].astype(jnp.bfloat16), w_ref[...].astype(jnp.bfloat16),
                preferred_element_type=jnp.float32)
    a = y[:, :D]
    b = y[:, D:]
    z_ref[...] = jnp.where(a > 0.0, a, jnp.exp(a) - 1.0) * b


def _gated_matmul(x, wc, slab):
    nblk = NSLAB // BM
    return pl.pallas_call(
        _gate_body,
        grid=(nblk,),
        in_specs=[
            pl.BlockSpec((BM, D), lambda i, s=slab, n=nblk: (s * n + i, 0)),
            pl.BlockSpec((D, 2 * D), lambda i: (0, 0)),
        ],
        out_specs=pl.BlockSpec((BM, D), lambda i: (i, 0)),
        out_shape=jax.ShapeDtypeStruct((NSLAB, D), jnp.float32),
    )(x, wc)


def _sc_body(z_hbm, idx_hbm, zero_hbm, out_hbm,
             idx_v, zb0, zb1, stage, shared, sem0, sem1):
    c = lax.axis_index("c")
    s = lax.axis_index("s")
    wid = c * NS + s
    gs = G // NS
    pltpu.sync_copy(zero_hbm.at[pl.ds(s * gs, gs)], shared.at[pl.ds(s * gs, gs)])
    pltpu.sync_copy(idx_hbm.at[wid], idx_v)
    plsc.subcore_barrier()

    row0 = wid * ROWS_W
    pltpu.make_async_copy(z_hbm.at[pl.ds(row0, CHUNK)], zb0, sem0).start()
    pltpu.make_async_copy(z_hbm.at[pl.ds(row0 + CHUNK, CHUNK)], zb1, sem1).start()

    def step(k, carry):
        j0 = 2 * k
        pltpu.make_async_copy(z_hbm.at[pl.ds(row0 + j0 * CHUNK, CHUNK)],
                              zb0, sem0).wait()
        pltpu.sync_copy(zb0, shared.at[idx_v.at[j0]], add=True)

        @pl.when(j0 + 2 < NCH)
        def _():
            pltpu.make_async_copy(
                z_hbm.at[pl.ds(row0 + (j0 + 2) * CHUNK, CHUNK)], zb0, sem0
            ).start()

        pltpu.make_async_copy(z_hbm.at[pl.ds(row0 + (j0 + 1) * CHUNK, CHUNK)],
                              zb1, sem1).wait()
        pltpu.sync_copy(zb1, shared.at[idx_v.at[j0 + 1]], add=True)

        @pl.when(j0 + 3 < NCH)
        def _():
            pltpu.make_async_copy(
                z_hbm.at[pl.ds(row0 + (j0 + 3) * CHUNK, CHUNK)], zb1, sem1
            ).start()

        return carry

    lax.fori_loop(0, NCH // 2, step, 0)
    plsc.subcore_barrier()
    pltpu.sync_copy(shared.at[pl.ds(s * gs, gs)], stage)
    pltpu.sync_copy(stage, out_hbm.at[c, pl.ds(s * gs, gs)])


def _segment_sum_sc(z, idx3, zeros):
    mesh = plsc.VectorSubcoreMesh(
        core_axis_name="c", subcore_axis_name="s",
        num_cores=NC, num_subcores=NS,
    )
    return pl.kernel(
        _sc_body,
        out_type=jax.ShapeDtypeStruct((NC, G, D), jnp.float32),
        mesh=mesh,
        scratch_types=[
            pltpu.VMEM((NCH, CHUNK), jnp.int32),
            pltpu.VMEM((CHUNK, D), jnp.float32),
            pltpu.VMEM((CHUNK, D), jnp.float32),
            pltpu.VMEM((G // NS, D), jnp.float32),
            pltpu.VMEM_SHARED((G, D), jnp.float32),
            pltpu.SemaphoreType.DMA,
            pltpu.SemaphoreType.DMA,
        ],
    )(z, idx3, zeros)


def _merge_body(p_ref, o_ref):
    o_ref[...] = jnp.sum(p_ref[...], axis=0)


def _merge(partials):
    return pl.pallas_call(
        _merge_body,
        out_shape=jax.ShapeDtypeStruct((G, D), jnp.float32),
    )(partials)


def kernel(input, graph_indices, node_counts, W1, W2):
    del node_counts
    wc = jnp.concatenate([W1, W2], axis=0).T
    idx4 = graph_indices.astype(jnp.int32).reshape(S, NW, NCH, CHUNK)
    zeros = jnp.zeros((G, D), jnp.float32)
    parts = []
    for slab in range(S):
        z = _gated_matmul(input, wc, slab)
        parts.append(_segment_sum_sc(z, idx4[slab], zeros))
    partials = jnp.stack(parts).reshape(S * NC, G, D)
    return _merge(partials)


# speedup vs baseline: 4.7677x; 1.0442x over previous
import jax
import jax.numpy as jnp
from jax import lax
from jax.experimental import pallas as pl
from jax.experimental.pallas import tpu as pltpu
from jax.experimental.pallas import tpu_sc as plsc

N = 320000
D = 128
G = 512
S = 5
NSLAB = N // S
NC, NS = 2, 16
NW = NC * NS
ROWS_W = NSLAB // NW
CHUNK = 80
NCH = ROWS_W // CHUNK
BM = 1600


def _gate_body(x_ref, w_ref, z_ref):
    y = jnp.dot(x_ref[...].astype(jnp.bfloat16), w_ref[...].astype(jnp.bfloat16),
                preferred_element_type=jnp.float32)
    a = y[:, :D]
    b = y[:, D:]
    z_ref[...] = jnp.where(a > 0.0, a, jnp.exp(a) - 1.0) * b


def _gated_matmul(x, wc, slab):
    nblk = NSLAB // BM
    return pl.pallas_call(
        _gate_body,
        grid=(nblk,),
        in_specs=[
            pl.BlockSpec((BM, D), lambda i, s=slab, n=nblk: (s * n + i, 0)),
            pl.BlockSpec((D, 2 * D), lambda i: (0, 0)),
        ],
        out_specs=pl.BlockSpec((BM, D), lambda i: (i, 0)),
        out_shape=jax.ShapeDtypeStruct((NSLAB, D), jnp.float32),
    )(x, wc)


def _sc_body(z_hbm, idx_hbm, zero_hbm, out_hbm,
             idx_v, zb0, zb1, stage, shared, sem0, sem1):
    c = lax.axis_index("c")
    s = lax.axis_index("s")
    wid = c * NS + s
    gs = G // NS
    pltpu.sync_copy(zero_hbm.at[pl.ds(s * gs, gs)], shared.at[pl.ds(s * gs, gs)])
    pltpu.sync_copy(idx_hbm.at[wid], idx_v)
    plsc.subcore_barrier()

    row0 = wid * ROWS_W
    pltpu.make_async_copy(z_hbm.at[pl.ds(row0, CHUNK)], zb0, sem0).start()
    pltpu.make_async_copy(z_hbm.at[pl.ds(row0 + CHUNK, CHUNK)], zb1, sem1).start()

    def step(k, carry):
        j0 = 2 * k
        pltpu.make_async_copy(z_hbm.at[pl.ds(row0 + j0 * CHUNK, CHUNK)],
                              zb0, sem0).wait()
        pltpu.sync_copy(zb0, shared.at[idx_v.at[j0]], add=True)

        @pl.when(j0 + 2 < NCH)
        def _():
            pltpu.make_async_copy(
                z_hbm.at[pl.ds(row0 + (j0 + 2) * CHUNK, CHUNK)], zb0, sem0
            ).start()

        pltpu.make_async_copy(z_hbm.at[pl.ds(row0 + (j0 + 1) * CHUNK, CHUNK)],
                              zb1, sem1).wait()
        pltpu.sync_copy(zb1, shared.at[idx_v.at[j0 + 1]], add=True)

        @pl.when(j0 + 3 < NCH)
        def _():
            pltpu.make_async_copy(
                z_hbm.at[pl.ds(row0 + (j0 + 3) * CHUNK, CHUNK)], zb1, sem1
            ).start()

        return carry

    lax.fori_loop(0, NCH // 2, step, 0)
    if NCH % 2:
        jt = NCH - 1
        pltpu.make_async_copy(z_hbm.at[pl.ds(row0 + jt * CHUNK, CHUNK)],
                              zb0, sem0).wait()
        pltpu.sync_copy(zb0, shared.at[idx_v.at[jt]], add=True)
    plsc.subcore_barrier()
    pltpu.sync_copy(shared.at[pl.ds(s * gs, gs)], stage)
    pltpu.sync_copy(stage, out_hbm.at[c, pl.ds(s * gs, gs)])


def _segment_sum_sc(z, idx3, zeros):
    mesh = plsc.VectorSubcoreMesh(
        core_axis_name="c", subcore_axis_name="s",
        num_cores=NC, num_subcores=NS,
    )
    return pl.kernel(
        _sc_body,
        out_type=jax.ShapeDtypeStruct((NC, G, D), jnp.float32),
        mesh=mesh,
        scratch_types=[
            pltpu.VMEM((NCH, CHUNK), jnp.int32),
            pltpu.VMEM((CHUNK, D), jnp.float32),
            pltpu.VMEM((CHUNK, D), jnp.float32),
            pltpu.VMEM((G // NS, D), jnp.float32),
            pltpu.VMEM_SHARED((G, D), jnp.float32),
            pltpu.SemaphoreType.DMA,
            pltpu.SemaphoreType.DMA,
        ],
    )(z, idx3, zeros)


def _merge_body(*refs):
    o_ref = refs[-1]
    acc = refs[0][0] + refs[0][1]
    for r in refs[1:-1]:
        acc = acc + r[0] + r[1]
    o_ref[...] = acc


def _merge(parts):
    return pl.pallas_call(
        _merge_body,
        out_shape=jax.ShapeDtypeStruct((G, D), jnp.float32),
    )(*parts)


def kernel(input, graph_indices, node_counts, W1, W2):
    del node_counts
    wc = jnp.concatenate([W1, W2], axis=0).T
    idx4 = graph_indices.astype(jnp.int32).reshape(S, NW, NCH, CHUNK)
    zeros = jnp.zeros((G, D), jnp.float32)
    parts = []
    for slab in range(S):
        z = _gated_matmul(input, wc, slab)
        parts.append(_segment_sum_sc(z, idx4[slab], zeros))
    return _merge(parts)
